# baseline (speedup 1.0000x reference)
import jax
import jax.numpy as jnp
from jax import lax
from jax.experimental import pallas as pl
from jax.experimental.pallas import tpu as pltpu
from jax.experimental.pallas import tpu_sc as plsc

D = 128
N = 10000
E = 320000
NC, NS = 2, 16
NW = NC * NS
K = 128
CHUNKS = 79
EPAD = NW * CHUNKS * K
PHASES = 2
PCH = (40, 39)
NPAD = 10112
ZROWS = 632
MMB = 2000


def _mm_body(x_ref, wt_ref, b_ref, hs_ref, mg_ref):
    y = jnp.dot(x_ref[...], wt_ref[...], preferred_element_type=jnp.float32)
    y = y + b_ref[...]
    hs_ref[...] = y[:, :D]
    mg_ref[...] = y[:, D:]


def _fin_body(hs_ref, p_ref, o_ref):
    o_ref[...] = jnp.maximum(hs_ref[...] + p_ref[0] + p_ref[1], 0.0)


def _sc_scatter_body(src_hbm, dst_hbm, msg_hbm, out_hbm,
                     idxs_v, idxd_v, rows_v, acc_sh, gsems, ssems):
    cid = lax.axis_index("c")
    sid = lax.axis_index("s")
    wid = sid * NC + cid

    def _zb(i, carry):
        r = i // 8
        c = (i % 8) * 16
        rows_v[0, r, pl.ds(c, 16)] = jnp.zeros((16,), jnp.float32)
        return carry
    lax.fori_loop(0, 1024, _zb, 0)
    zbase = sid * ZROWS
    for t in range(4):
        pltpu.sync_copy(rows_v.at[0], acc_sh.at[pl.ds(zbase + t * 128, 128)])
    pltpu.sync_copy(rows_v.at[0, pl.ds(0, ZROWS - 512)],
                    acc_sh.at[pl.ds(zbase + 512, ZROWS - 512)])
    plsc.subcore_barrier()

    def _gather(j, b):
        pltpu.async_copy(msg_hbm.at[idxs_v.at[j]], rows_v.at[b], gsems.at[b])

    def _gwait(j, b):
        pltpu.make_async_copy(msg_hbm.at[idxs_v.at[j]], rows_v.at[b],
                              gsems.at[b]).wait()

    def _scat(j, b):
        pltpu.async_copy(rows_v.at[b], acc_sh.at[idxd_v.at[j]], ssems.at[b],
                         add=True)

    def _swait(j, b):
        pltpu.make_async_copy(rows_v.at[b], acc_sh.at[idxd_v.at[j]],
                              ssems.at[b]).wait()

    for p in range(PHASES):
        nc = PCH[p]
        pltpu.sync_copy(src_hbm.at[pl.ds(p * PCH[0], nc), wid],
                        idxs_v.at[pl.ds(0, nc)])
        pltpu.sync_copy(dst_hbm.at[pl.ds(p * PCH[0], nc), wid],
                        idxd_v.at[pl.ds(0, nc)])
        _gather(0, 0)
        _gather(1, 1)

        def _pair(i, carry):
            j0 = i * 2
            _gwait(j0, 0)
            _scat(j0, 0)
            _gwait(j0 + 1, 1)
            _scat(j0 + 1, 1)
            _swait(j0, 0)

            @pl.when(j0 + 2 < nc)
            def _():
                _gather(j0 + 2, 0)
            _swait(j0 + 1, 1)

            @pl.when(j0 + 3 < nc)
            def _():
                _gather(j0 + 3, 1)
            return carry
        lax.fori_loop(0, nc // 2, _pair, 0)
        if nc % 2 == 1:
            _gwait(nc - 1, 0)
            _scat(nc - 1, 0)
            _swait(nc - 1, 0)
    plsc.subcore_barrier()

    pltpu.sync_copy(acc_sh.at[pl.ds(zbase, ZROWS)],
                    out_hbm.at[cid, pl.ds(zbase, ZROWS)])


@jax.jit
def _sc_scatter(src, dst, msg):
    mesh = plsc.VectorSubcoreMesh(core_axis_name="c", subcore_axis_name="s",
                                  num_cores=NC, num_subcores=NS)
    f = pl.kernel(
        _sc_scatter_body,
        out_type=jax.ShapeDtypeStruct((NC, NPAD, D), jnp.float32),
        mesh=mesh,
        scratch_types=[
            pltpu.VMEM((PCH[0], K), jnp.int32),
            pltpu.VMEM((PCH[0], K), jnp.int32),
            pltpu.VMEM((2, K, D), jnp.float32),
            pltpu.VMEM_SHARED((NPAD, D), jnp.float32),
            pltpu.SemaphoreType.DMA((2,)),
            pltpu.SemaphoreType.DMA((2,)),
        ],
        compiler_params=pltpu.CompilerParams(use_tc_tiling_on_sc=False),
    )
    return f(src, dst, msg)


@jax.jit
def _mm(x, wt, b):
    return pl.pallas_call(
        _mm_body,
        grid=(N // MMB,),
        in_specs=[
            pl.BlockSpec((MMB, D), lambda i: (i, 0)),
            pl.BlockSpec((D, 2 * D), lambda i: (0, 0)),
            pl.BlockSpec((1, 2 * D), lambda i: (0, 0)),
        ],
        out_specs=[
            pl.BlockSpec((MMB, D), lambda i: (i, 0)),
            pl.BlockSpec((MMB, D), lambda i: (i, 0)),
        ],
        out_shape=[
            jax.ShapeDtypeStruct((N, D), jnp.float32),
            jax.ShapeDtypeStruct((N, D), jnp.float32),
        ],
    )(x, wt, b)


@jax.jit
def _finalize(hs, p):
    return pl.pallas_call(
        _fin_body,
        grid=(N // MMB,),
        in_specs=[
            pl.BlockSpec((MMB, D), lambda i: (i, 0)),
            pl.BlockSpec((NC, MMB, D), lambda i: (0, i, 0)),
        ],
        out_specs=pl.BlockSpec((MMB, D), lambda i: (i, 0)),
        out_shape=jax.ShapeDtypeStruct((N, D), jnp.float32),
    )(hs, p)


def kernel(x, edge_index, W_self, b_self, W_neigh, b_neigh):
    src = edge_index[0].astype(jnp.int32)
    dst = edge_index[1].astype(jnp.int32)
    pad = EPAD - E
    pad_src = jnp.arange(pad, dtype=jnp.int32) % N
    pad_dst = N + jnp.arange(pad, dtype=jnp.int32) % (NPAD - N)
    src_p = jnp.concatenate([src, pad_src]).reshape(CHUNKS, NW, K)
    dst_p = jnp.concatenate([dst, pad_dst]).reshape(CHUNKS, NW, K)
    wt = jnp.concatenate([W_self.T, W_neigh.T], axis=1)
    b = jnp.concatenate([b_self, b_neigh]).reshape(1, 2 * D)
    hs, msg = _mm(x, wt, b)
    partials = _sc_scatter(src_p, dst_p, msg)
    return _finalize(hs, partials)


# speedup vs baseline: 1.0194x; 1.0194x over previous
import jax
import jax.numpy as jnp
from jax import lax
from jax.experimental import pallas as pl
from jax.experimental.pallas import tpu as pltpu
from jax.experimental.pallas import tpu_sc as plsc

D = 128
N = 10000
E = 320000
NC, NS = 2, 16
NW = NC * NS
K = 128
CHUNKS = 79
EPAD = NW * CHUNKS * K
PHASES = 2
PCH = (40, 39)
NPAD = 10112
ZROWS = 632
MMB = 2000


def _mm_body(x_ref, wt_ref, b_ref, mg_ref):
    y = jnp.dot(x_ref[...], wt_ref[...], preferred_element_type=jnp.float32)
    mg_ref[...] = y + b_ref[...]


def _fin_body(hs_ref, p_ref, o_ref):
    o_ref[...] = jnp.maximum(hs_ref[...] + p_ref[0] + p_ref[1], 0.0)


def _sc_scatter_body(src_hbm, dst_hbm, msg_hbm, out_hbm,
                     idxs_v, idxd_v, rows_v, acc_sh, gsems, ssems):
    cid = lax.axis_index("c")
    sid = lax.axis_index("s")
    wid = sid * NC + cid

    def _zb(i, carry):
        r = i // 8
        c = (i % 8) * 16
        rows_v[0, r, pl.ds(c, 16)] = jnp.zeros((16,), jnp.float32)
        return carry
    lax.fori_loop(0, 1024, _zb, 0)
    zbase = sid * ZROWS
    for t in range(4):
        pltpu.sync_copy(rows_v.at[0], acc_sh.at[pl.ds(zbase + t * 128, 128)])
    pltpu.sync_copy(rows_v.at[0, pl.ds(0, ZROWS - 512)],
                    acc_sh.at[pl.ds(zbase + 512, ZROWS - 512)])
    plsc.subcore_barrier()

    def _gather(j, b):
        pltpu.async_copy(msg_hbm.at[idxs_v.at[j]], rows_v.at[b], gsems.at[b])

    def _gwait(j, b):
        pltpu.make_async_copy(msg_hbm.at[idxs_v.at[j]], rows_v.at[b],
                              gsems.at[b]).wait()

    def _scat(j, b):
        pltpu.async_copy(rows_v.at[b], acc_sh.at[idxd_v.at[j]], ssems.at[b],
                         add=True)

    def _swait(j, b):
        pltpu.make_async_copy(rows_v.at[b], acc_sh.at[idxd_v.at[j]],
                              ssems.at[b]).wait()

    for p in range(PHASES):
        nc = PCH[p]
        pltpu.sync_copy(src_hbm.at[pl.ds(p * PCH[0], nc), wid],
                        idxs_v.at[pl.ds(0, nc)])
        pltpu.sync_copy(dst_hbm.at[pl.ds(p * PCH[0], nc), wid],
                        idxd_v.at[pl.ds(0, nc)])
        _gather(0, 0)
        _gather(1, 1)

        def _pair(i, carry):
            j0 = i * 2
            _gwait(j0, 0)
            _scat(j0, 0)
            _gwait(j0 + 1, 1)
            _scat(j0 + 1, 1)
            _swait(j0, 0)

            @pl.when(j0 + 2 < nc)
            def _():
                _gather(j0 + 2, 0)
            _swait(j0 + 1, 1)

            @pl.when(j0 + 3 < nc)
            def _():
                _gather(j0 + 3, 1)
            return carry
        lax.fori_loop(0, nc // 2, _pair, 0)
        if nc % 2 == 1:
            _gwait(nc - 1, 0)
            _scat(nc - 1, 0)
            _swait(nc - 1, 0)
    plsc.subcore_barrier()

    pltpu.sync_copy(acc_sh.at[pl.ds(zbase, ZROWS)],
                    out_hbm.at[cid, pl.ds(zbase, ZROWS)])


@jax.jit
def _sc_scatter(src, dst, msg):
    mesh = plsc.VectorSubcoreMesh(core_axis_name="c", subcore_axis_name="s",
                                  num_cores=NC, num_subcores=NS)
    f = pl.kernel(
        _sc_scatter_body,
        out_type=jax.ShapeDtypeStruct((NC, NPAD, D), jnp.float32),
        mesh=mesh,
        scratch_types=[
            pltpu.VMEM((PCH[0], K), jnp.int32),
            pltpu.VMEM((PCH[0], K), jnp.int32),
            pltpu.VMEM((2, K, D), jnp.float32),
            pltpu.VMEM_SHARED((NPAD, D), jnp.float32),
            pltpu.SemaphoreType.DMA((2,)),
            pltpu.SemaphoreType.DMA((2,)),
        ],
        compiler_params=pltpu.CompilerParams(use_tc_tiling_on_sc=False),
    )
    return f(src, dst, msg)


@jax.jit
def _mm(x, wt, b):
    return pl.pallas_call(
        _mm_body,
        grid=(N // MMB,),
        in_specs=[
            pl.BlockSpec((MMB, D), lambda i: (i, 0)),
            pl.BlockSpec((D, D), lambda i: (0, 0)),
            pl.BlockSpec((1, D), lambda i: (0, 0)),
        ],
        out_specs=pl.BlockSpec((MMB, D), lambda i: (i, 0)),
        out_shape=jax.ShapeDtypeStruct((N, D), jnp.float32),
    )(x, wt, b)


@jax.jit
def _finalize(hs, p):
    return pl.pallas_call(
        _fin_body,
        grid=(N // MMB,),
        in_specs=[
            pl.BlockSpec((MMB, D), lambda i: (i, 0)),
            pl.BlockSpec((NC, MMB, D), lambda i: (0, i, 0)),
        ],
        out_specs=pl.BlockSpec((MMB, D), lambda i: (i, 0)),
        out_shape=jax.ShapeDtypeStruct((N, D), jnp.float32),
    )(hs, p)


def kernel(x, edge_index, W_self, b_self, W_neigh, b_neigh):
    src = edge_index[0].astype(jnp.int32)
    dst = edge_index[1].astype(jnp.int32)
    pad = EPAD - E
    pad_src = jnp.arange(pad, dtype=jnp.int32) % N
    pad_dst = N + jnp.arange(pad, dtype=jnp.int32) % (NPAD - N)
    src_p = jnp.concatenate([src, pad_src]).reshape(CHUNKS, NW, K)
    dst_p = jnp.concatenate([dst, pad_dst]).reshape(CHUNKS, NW, K)
    msg = _mm(x, W_neigh.T, b_neigh.reshape(1, D))
    partials = _sc_scatter(src_p, dst_p, msg)
    hs = _mm(x, W_self.T, b_self.reshape(1, D))
    return _finalize(hs, partials)


# speedup vs baseline: 1.1405x; 1.1188x over previous
import jax
import jax.numpy as jnp
from jax import lax
from jax.experimental import pallas as pl
from jax.experimental.pallas import tpu as pltpu
from jax.experimental.pallas import tpu_sc as plsc

D = 128
N = 10000
E = 320000
NC, NS = 2, 16
NW = NC * NS
K = 112
CHUNKS = 90
EPAD = NW * CHUNKS * K
PHASES = 3
PCH = 30
NBUF = 3
NPAD = 10112
ZROWS = 632
MMB = 2000


def _mm_body(x_ref, wt_ref, b_ref, mg_ref):
    y = jnp.dot(x_ref[...], wt_ref[...], preferred_element_type=jnp.float32)
    mg_ref[...] = y + b_ref[...]


def _fin_body(hs_ref, p_ref, o_ref):
    o_ref[...] = jnp.maximum(hs_ref[...] + p_ref[0] + p_ref[1], 0.0)


def _sc_scatter_body(src_hbm, dst_hbm, msg_hbm, out_hbm,
                     idxs_v, idxd_v, rows_v, acc_sh, gsems, ssems):
    cid = lax.axis_index("c")
    sid = lax.axis_index("s")
    wid = sid * NC + cid

    def _zb(i, carry):
        r = i // 8
        c = (i % 8) * 16
        rows_v[0, r, pl.ds(c, 16)] = jnp.zeros((16,), jnp.float32)
        return carry
    lax.fori_loop(0, K * 8, _zb, 0)
    zbase = sid * ZROWS
    for t in range(ZROWS // K):
        pltpu.sync_copy(rows_v.at[0], acc_sh.at[pl.ds(zbase + t * K, K)])
    pltpu.sync_copy(rows_v.at[0, pl.ds(0, ZROWS % K)],
                    acc_sh.at[pl.ds(zbase + (ZROWS // K) * K, ZROWS % K)])
    plsc.subcore_barrier()

    def _gather(j, b):
        pltpu.async_copy(msg_hbm.at[idxs_v.at[j]], rows_v.at[b], gsems.at[b])

    def _gwait(j, b):
        pltpu.make_async_copy(msg_hbm.at[idxs_v.at[j]], rows_v.at[b],
                              gsems.at[b]).wait()

    def _scat(j, b):
        pltpu.async_copy(rows_v.at[b], acc_sh.at[idxd_v.at[j]], ssems.at[b],
                         add=True)

    def _swait(j, b):
        pltpu.make_async_copy(rows_v.at[b], acc_sh.at[idxd_v.at[j]],
                              ssems.at[b]).wait()

    for p in range(PHASES):
        pltpu.sync_copy(src_hbm.at[pl.ds(p * PCH, PCH), wid], idxs_v)
        pltpu.sync_copy(dst_hbm.at[pl.ds(p * PCH, PCH), wid], idxd_v)
        for b in range(NBUF):
            _gather(b, b)

        def _grp(i, carry):
            j0 = i * NBUF
            for b in range(NBUF):
                _gwait(j0 + b, b)
                _scat(j0 + b, b)
            for b in range(NBUF):
                _swait(j0 + b, b)

                @pl.when(j0 + NBUF + b < PCH)
                def _():
                    _gather(j0 + NBUF + b, b)
            return carry
        lax.fori_loop(0, PCH // NBUF, _grp, 0)
    plsc.subcore_barrier()

    pltpu.sync_copy(acc_sh.at[pl.ds(zbase, ZROWS)],
                    out_hbm.at[cid, pl.ds(zbase, ZROWS)])


@jax.jit
def _sc_scatter(src, dst, msg):
    mesh = plsc.VectorSubcoreMesh(core_axis_name="c", subcore_axis_name="s",
                                  num_cores=NC, num_subcores=NS)
    f = pl.kernel(
        _sc_scatter_body,
        out_type=jax.ShapeDtypeStruct((NC, NPAD, D), jnp.float32),
        mesh=mesh,
        scratch_types=[
            pltpu.VMEM((PCH, K), jnp.int32),
            pltpu.VMEM((PCH, K), jnp.int32),
            pltpu.VMEM((NBUF, K, D), jnp.float32),
            pltpu.VMEM_SHARED((NPAD, D), jnp.float32),
            pltpu.SemaphoreType.DMA((NBUF,)),
            pltpu.SemaphoreType.DMA((NBUF,)),
        ],
        compiler_params=pltpu.CompilerParams(use_tc_tiling_on_sc=False),
    )
    return f(src, dst, msg)


@jax.jit
def _mm(x, wt, b):
    return pl.pallas_call(
        _mm_body,
        grid=(N // MMB,),
        in_specs=[
            pl.BlockSpec((MMB, D), lambda i: (i, 0)),
            pl.BlockSpec((D, D), lambda i: (0, 0)),
            pl.BlockSpec((1, D), lambda i: (0, 0)),
        ],
        out_specs=pl.BlockSpec((MMB, D), lambda i: (i, 0)),
        out_shape=jax.ShapeDtypeStruct((N, D), jnp.float32),
    )(x, wt, b)


@jax.jit
def _finalize(hs, p):
    return pl.pallas_call(
        _fin_body,
        grid=(N // MMB,),
        in_specs=[
            pl.BlockSpec((MMB, D), lambda i: (i, 0)),
            pl.BlockSpec((NC, MMB, D), lambda i: (0, i, 0)),
        ],
        out_specs=pl.BlockSpec((MMB, D), lambda i: (i, 0)),
        out_shape=jax.ShapeDtypeStruct((N, D), jnp.float32),
    )(hs, p)


def kernel(x, edge_index, W_self, b_self, W_neigh, b_neigh):
    src = edge_index[0].astype(jnp.int32)
    dst = edge_index[1].astype(jnp.int32)
    pad = EPAD - E
    pad_src = jnp.arange(pad, dtype=jnp.int32) % N
    pad_dst = N + jnp.arange(pad, dtype=jnp.int32) % (NPAD - N)
    src_p = jnp.concatenate([src, pad_src]).reshape(CHUNKS, NW, K)
    dst_p = jnp.concatenate([dst, pad_dst]).reshape(CHUNKS, NW, K)
    msg = _mm(x, W_neigh.T, b_neigh.reshape(1, D))
    partials = _sc_scatter(src_p, dst_p, msg)
    hs = _mm(x, W_self.T, b_self.reshape(1, D))
    return _finalize(hs, partials)


# speedup vs baseline: 1.1508x; 1.0090x over previous
import jax
import jax.numpy as jnp
from jax import lax
from jax.experimental import pallas as pl
from jax.experimental.pallas import tpu as pltpu
from jax.experimental.pallas import tpu_sc as plsc

D = 128
N = 10000
E = 320000
NC, NS = 2, 16
NW = NC * NS
K = 112
CHUNKS = 90
EPAD = NW * CHUNKS * K
PHASES = 3
PCH = 30
NBUF = 3
NPAD = 10112
ZROWS = 632
MMB = 2000


def _mm_body(x_ref, wt_ref, b_ref, mg_ref):
    y = jnp.dot(x_ref[...], wt_ref[...], preferred_element_type=jnp.float32)
    mg_ref[...] = y + b_ref[...]


def _fin_body(hs_ref, p_ref, o_ref):
    o_ref[...] = jnp.maximum(hs_ref[...] + p_ref[0] + p_ref[1], 0.0)


def _sc_scatter_body(src_hbm, dst_hbm, msg_hbm, out_hbm,
                     idxs_v, idxd_v, rows_v, acc_sh, gsems, ssems):
    cid = lax.axis_index("c")
    sid = lax.axis_index("s")
    wid = sid * NC + cid

    pltpu.async_copy(src_hbm.at[pl.ds(0, PCH), wid], idxs_v, gsems.at[0])
    pltpu.async_copy(dst_hbm.at[pl.ds(0, PCH), wid], idxd_v, gsems.at[1])

    def _zb(i, carry):
        r = i // 8
        c = (i % 8) * 16
        rows_v[0, r, pl.ds(c, 16)] = jnp.zeros((16,), jnp.float32)
        return carry
    lax.fori_loop(0, K * 8, _zb, 0)
    zbase = sid * ZROWS
    for t in range(ZROWS // K):
        pltpu.sync_copy(rows_v.at[0], acc_sh.at[pl.ds(zbase + t * K, K)])
    pltpu.sync_copy(rows_v.at[0, pl.ds(0, ZROWS % K)],
                    acc_sh.at[pl.ds(zbase + (ZROWS // K) * K, ZROWS % K)])
    plsc.subcore_barrier()

    def _gather(j, b):
        pltpu.async_copy(msg_hbm.at[idxs_v.at[j]], rows_v.at[b], gsems.at[b])

    def _gwait(j, b):
        pltpu.make_async_copy(msg_hbm.at[idxs_v.at[j]], rows_v.at[b],
                              gsems.at[b]).wait()

    def _scat(j, b):
        pltpu.async_copy(rows_v.at[b], acc_sh.at[idxd_v.at[j]], ssems.at[b],
                         add=True)

    def _swait(j, b):
        pltpu.make_async_copy(rows_v.at[b], acc_sh.at[idxd_v.at[j]],
                              ssems.at[b]).wait()

    for p in range(PHASES):
        if p == 0:
            pltpu.make_async_copy(src_hbm.at[pl.ds(0, PCH), wid], idxs_v,
                                  gsems.at[0]).wait()
            pltpu.make_async_copy(dst_hbm.at[pl.ds(0, PCH), wid], idxd_v,
                                  gsems.at[1]).wait()
        else:
            pltpu.sync_copy(src_hbm.at[pl.ds(p * PCH, PCH), wid], idxs_v)
            pltpu.sync_copy(dst_hbm.at[pl.ds(p * PCH, PCH), wid], idxd_v)
        for b in range(NBUF):
            _gather(b, b)

        def _grp(i, carry):
            j0 = i * NBUF
            for b in range(NBUF):
                _gwait(j0 + b, b)
                _scat(j0 + b, b)
            for b in range(NBUF):
                _swait(j0 + b, b)

                @pl.when(j0 + NBUF + b < PCH)
                def _():
                    _gather(j0 + NBUF + b, b)
            return carry
        lax.fori_loop(0, PCH // NBUF, _grp, 0)
    plsc.subcore_barrier()

    pltpu.sync_copy(acc_sh.at[pl.ds(zbase, ZROWS)],
                    out_hbm.at[cid, pl.ds(zbase, ZROWS)])


@jax.jit
def _sc_scatter(src, dst, msg):
    mesh = plsc.VectorSubcoreMesh(core_axis_name="c", subcore_axis_name="s",
                                  num_cores=NC, num_subcores=NS)
    f = pl.kernel(
        _sc_scatter_body,
        out_type=jax.ShapeDtypeStruct((NC, NPAD, D), jnp.float32),
        mesh=mesh,
        scratch_types=[
            pltpu.VMEM((PCH, K), jnp.int32),
            pltpu.VMEM((PCH, K), jnp.int32),
            pltpu.VMEM((NBUF, K, D), jnp.float32),
            pltpu.VMEM_SHARED((NPAD, D), jnp.float32),
            pltpu.SemaphoreType.DMA((NBUF,)),
            pltpu.SemaphoreType.DMA((NBUF,)),
        ],
        compiler_params=pltpu.CompilerParams(use_tc_tiling_on_sc=False),
    )
    return f(src, dst, msg)


@jax.jit
def _mm(x, wt, b):
    return pl.pallas_call(
        _mm_body,
        grid=(N // MMB,),
        in_specs=[
            pl.BlockSpec((MMB, D), lambda i: (i, 0)),
            pl.BlockSpec((D, D), lambda i: (0, 0)),
            pl.BlockSpec((1, D), lambda i: (0, 0)),
        ],
        out_specs=pl.BlockSpec((MMB, D), lambda i: (i, 0)),
        out_shape=jax.ShapeDtypeStruct((N, D), jnp.float32),
    )(x, wt, b)


@jax.jit
def _finalize(hs, p):
    return pl.pallas_call(
        _fin_body,
        grid=(N // MMB,),
        in_specs=[
            pl.BlockSpec((MMB, D), lambda i: (i, 0)),
            pl.BlockSpec((NC, MMB, D), lambda i: (0, i, 0)),
        ],
        out_specs=pl.BlockSpec((MMB, D), lambda i: (i, 0)),
        out_shape=jax.ShapeDtypeStruct((N, D), jnp.float32),
    )(hs, p)


def kernel(x, edge_index, W_self, b_self, W_neigh, b_neigh):
    src = edge_index[0].astype(jnp.int32)
    dst = edge_index[1].astype(jnp.int32)
    pad = EPAD - E
    pad_src = jnp.arange(pad, dtype=jnp.int32) % N
    pad_dst = N + jnp.arange(pad, dtype=jnp.int32) % (NPAD - N)
    src_p = jnp.concatenate([src, pad_src]).reshape(CHUNKS, NW, K)
    dst_p = jnp.concatenate([dst, pad_dst]).reshape(CHUNKS, NW, K)
    msg = _mm(x, W_neigh.T, b_neigh.reshape(1, D))
    partials = _sc_scatter(src_p, dst_p, msg)
    hs = _mm(x, W_self.T, b_self.reshape(1, D))
    return _finalize(hs, partials)
